# Initial kernel scaffold; baseline (speedup 1.0000x reference)
import jax
import jax.numpy as jnp
from jax.experimental import pallas as pl


def kernel(x, edge_index, etypes, norm, weight, w_comp, h_bias, loop_weight):
    raise NotImplementedError("write your pallas kernel here")


# speedup vs baseline: 16.8211x; 16.8211x over previous
import functools

import jax
import jax.numpy as jnp
from jax import lax
from jax.experimental import pallas as pl
from jax.experimental.pallas import tpu as pltpu
from jax.experimental.pallas import tpu_sc as plsc

N = 10000
E = 320000
IN = 128
OUT = 128
R = 8
B = 4

NC = 2
NS = 16
LANES = 16
NW = NC * NS
EPW = E // NW
CH = 80
KCH = EPW // CH
ROWS_PT = N // NS
ZROWS = 25
XBLK = 400


def _xw_body(wc_ref, w_ref, x_ref, o_ref):
    xb = x_ref[...]
    prj = [jnp.dot(xb, w_ref[b], preferred_element_type=jnp.float32)
           for b in range(B)]
    for r in range(R):
        acc = prj[0] * wc_ref[r, 0]
        for b in range(1, B):
            acc = acc + prj[b] * wc_ref[r, b]
        o_ref[:, r * OUT:(r + 1) * OUT] = acc


def _combine_body(lw_ref, bias_ref, x_ref, p_ref, o_ref):
    acc = jnp.dot(x_ref[...], lw_ref[...], preferred_element_type=jnp.float32)
    o_ref[...] = acc + p_ref[0] + p_ref[1] + bias_ref[...]


def _sc_body(xw_hbm, idx_hbm, dst_hbm, nrm_hbm, out_hbm,
             idx_v, dst_v, nrm_v, rows_v, zb_v, acc_sh):
    c = lax.axis_index("c")
    s = lax.axis_index("s")
    wid = c * NS + s

    z16 = jnp.zeros((LANES,), jnp.float32)

    @pl.loop(0, ZROWS)
    def _(i):
        for j in range(OUT // LANES):
            zb_v[i, pl.ds(j * LANES, LANES)] = z16

    for k in range(ROWS_PT // ZROWS):
        pltpu.sync_copy(zb_v, acc_sh.at[pl.ds(s * ROWS_PT + k * ZROWS, ZROWS)])
    plsc.subcore_barrier()

    pltpu.sync_copy(idx_hbm.at[pl.ds(wid * KCH, KCH)], idx_v)
    pltpu.sync_copy(dst_hbm.at[pl.ds(wid * KCH, KCH)], dst_v)
    pltpu.sync_copy(nrm_hbm.at[pl.ds(wid * KCH, KCH)], nrm_v)

    @pl.loop(0, KCH)
    def _(k):
        pltpu.sync_copy(xw_hbm.at[idx_v.at[k]], rows_v)

        @pl.loop(0, CH)
        def _(e):
            nb = plsc.load_gather(
                nrm_v, [jnp.full((LANES,), k, jnp.int32),
                        jnp.full((LANES,), e, jnp.int32)])
            for j in range(OUT // LANES):
                sl = (e, pl.ds(j * LANES, LANES))
                rows_v[sl] = rows_v[sl] * nb

        pltpu.sync_copy(rows_v, acc_sh.at[dst_v.at[k]], add=True)

    plsc.subcore_barrier()
    pltpu.sync_copy(acc_sh.at[pl.ds(s * ROWS_PT, ROWS_PT)],
                    out_hbm.at[c, pl.ds(s * ROWS_PT, ROWS_PT)])


def _sc_scatter(xw_flat, flat_idx, dst_blk, nrm_blk):
    mesh = plsc.VectorSubcoreMesh(core_axis_name="c", subcore_axis_name="s")
    f = pl.kernel(
        _sc_body,
        out_type=jax.ShapeDtypeStruct((NC, N, OUT), jnp.float32),
        mesh=mesh,
        compiler_params=pltpu.CompilerParams(
            use_tc_tiling_on_sc=False, needs_layout_passes=False),
        scratch_types=[
            pltpu.VMEM((KCH, CH), jnp.int32),
            pltpu.VMEM((KCH, CH), jnp.int32),
            pltpu.VMEM((KCH, CH), jnp.float32),
            pltpu.VMEM((CH, OUT), jnp.float32),
            pltpu.VMEM((ZROWS, OUT), jnp.float32),
            pltpu.VMEM_SHARED((N, OUT), jnp.float32),
        ],
    )
    return f(xw_flat, flat_idx, dst_blk, nrm_blk)


def kernel(x, edge_index, etypes, norm, weight, w_comp, h_bias, loop_weight):
    x = x.astype(jnp.float32)
    src = edge_index[0].astype(jnp.int32)
    dst = edge_index[1].astype(jnp.int32)
    et = etypes.astype(jnp.int32)
    flat_idx = (src * R + et).reshape(E // CH, CH)
    dst_blk = dst.reshape(E // CH, CH)
    nrm_blk = norm.astype(jnp.float32).reshape(E // CH, CH)

    xw = pl.pallas_call(
        _xw_body,
        grid=(N // XBLK,),
        in_specs=[
            pl.BlockSpec(memory_space=pltpu.SMEM),
            pl.BlockSpec((B, IN, OUT), lambda i: (0, 0, 0)),
            pl.BlockSpec((XBLK, IN), lambda i: (i, 0)),
        ],
        out_specs=pl.BlockSpec((XBLK, R * OUT), lambda i: (i, 0)),
        out_shape=jax.ShapeDtypeStruct((N, R * OUT), jnp.float32),
    )(w_comp, weight, x)

    parts = _sc_scatter(xw.reshape(N * R, OUT), flat_idx, dst_blk, nrm_blk)

    h = pl.pallas_call(
        _combine_body,
        grid=(N // XBLK,),
        in_specs=[
            pl.BlockSpec((IN, OUT), lambda i: (0, 0)),
            pl.BlockSpec((1, OUT), lambda i: (0, 0)),
            pl.BlockSpec((XBLK, IN), lambda i: (i, 0)),
            pl.BlockSpec((NC, XBLK, OUT), lambda i: (0, i, 0)),
        ],
        out_specs=pl.BlockSpec((XBLK, OUT), lambda i: (i, 0)),
        out_shape=jax.ShapeDtypeStruct((N, OUT), jnp.float32),
    )(loop_weight, h_bias.reshape(1, OUT), x, parts)

    return h


# speedup vs baseline: 27.9048x; 1.6589x over previous
import functools

import jax
import jax.numpy as jnp
from jax import lax
from jax.experimental import pallas as pl
from jax.experimental.pallas import tpu as pltpu
from jax.experimental.pallas import tpu_sc as plsc

N = 10000
E = 320000
IN = 128
OUT = 128
R = 8
B = 4

NC = 2
NS = 16
LANES = 16
NW = NC * NS
EPW = E // NW
CH = 80
KCH = EPW // CH
ROWS_PT = N // NS
ZROWS = 25
XBLK = 400


def _xw_body(wc_ref, w_ref, x_ref, o_ref):
    xb = x_ref[...]
    prj = [jnp.dot(xb, w_ref[b], preferred_element_type=jnp.float32)
           for b in range(B)]
    for r in range(R):
        acc = prj[0] * wc_ref[r, 0]
        for b in range(1, B):
            acc = acc + prj[b] * wc_ref[r, b]
        o_ref[:, r * OUT:(r + 1) * OUT] = acc


def _combine_body(lw_ref, bias_ref, x_ref, p_ref, o_ref):
    acc = jnp.dot(x_ref[...], lw_ref[...], preferred_element_type=jnp.float32)
    o_ref[...] = acc + p_ref[0] + p_ref[1] + bias_ref[...]


def _sc_body(xw_hbm, idx_hbm, dst_hbm, nrm_hbm, out_hbm,
             idx_v, dst_v, nrm_v, rows0, rows1, acc_sh,
             sg0, sg1, ss0, ss1):
    c = lax.axis_index("c")
    s = lax.axis_index("s")
    wid = c * NS + s

    dm0 = pltpu.async_copy(idx_hbm.at[pl.ds(wid * KCH, KCH)], idx_v, sg0)
    dm1 = pltpu.async_copy(dst_hbm.at[pl.ds(wid * KCH, KCH)], dst_v, sg1)
    dm2 = pltpu.async_copy(nrm_hbm.at[pl.ds(wid * KCH, KCH)], nrm_v, ss0)

    z16 = jnp.zeros((LANES,), jnp.float32)

    @pl.loop(0, ZROWS)
    def _(i):
        for j in range(OUT // LANES):
            rows1[i, pl.ds(j * LANES, LANES)] = z16

    for k in range(ROWS_PT // ZROWS):
        pltpu.async_copy(
            rows1.at[pl.ds(0, ZROWS)],
            acc_sh.at[pl.ds(s * ROWS_PT + k * ZROWS, ZROWS)], ss1)
    for k in range(ROWS_PT // ZROWS):
        pltpu.make_async_copy(
            rows1.at[pl.ds(0, ZROWS)],
            acc_sh.at[pl.ds(s * ROWS_PT, ZROWS)], ss1).wait()
    dm0.wait()
    dm1.wait()
    dm2.wait()
    plsc.subcore_barrier()

    def scale(buf, k):
        @plsc.parallel_loop(0, CH, unroll=2)
        def _(e):
            nb = plsc.load_gather(
                nrm_v, [jnp.full((LANES,), k, jnp.int32),
                        jnp.full((LANES,), e, jnp.int32)])
            for j in range(OUT // LANES):
                sl = (e, pl.ds(j * LANES, LANES))
                buf[sl] = buf[sl] * nb

    pltpu.async_copy(xw_hbm.at[idx_v.at[0]], rows0, sg0)

    @pl.loop(0, KCH - 1, step=2)
    def _(kk):
        @pl.when(kk > 0)
        def _():
            pltpu.make_async_copy(
                rows1, acc_sh.at[dst_v.at[kk - 1]], ss1).wait()
        dg1 = pltpu.async_copy(xw_hbm.at[idx_v.at[kk + 1]], rows1, sg1)
        pltpu.make_async_copy(xw_hbm.at[idx_v.at[kk]], rows0, sg0).wait()
        scale(rows0, kk)
        ds0 = pltpu.async_copy(rows0, acc_sh.at[dst_v.at[kk]], ss0, add=True)
        dg1.wait()
        scale(rows1, kk + 1)
        ds0.wait()
        pltpu.async_copy(xw_hbm.at[idx_v.at[kk + 2]], rows0, sg0)
        pltpu.async_copy(rows1, acc_sh.at[dst_v.at[kk + 1]], ss1, add=True)

    pltpu.make_async_copy(xw_hbm.at[idx_v.at[KCH - 1]], rows0, sg0).wait()
    scale(rows0, KCH - 1)
    pltpu.async_copy(rows0, acc_sh.at[dst_v.at[KCH - 1]], ss0, add=True)
    pltpu.make_async_copy(rows0, acc_sh.at[dst_v.at[KCH - 1]], ss0).wait()
    pltpu.make_async_copy(rows1, acc_sh.at[dst_v.at[KCH - 2]], ss1).wait()

    plsc.subcore_barrier()
    pltpu.sync_copy(acc_sh.at[pl.ds(s * ROWS_PT, ROWS_PT)],
                    out_hbm.at[c, pl.ds(s * ROWS_PT, ROWS_PT)])


def _sc_scatter(xw_flat, flat_idx, dst_blk, nrm_blk):
    mesh = plsc.VectorSubcoreMesh(core_axis_name="c", subcore_axis_name="s")
    f = pl.kernel(
        _sc_body,
        out_type=jax.ShapeDtypeStruct((NC, N, OUT), jnp.float32),
        mesh=mesh,
        compiler_params=pltpu.CompilerParams(
            use_tc_tiling_on_sc=False, needs_layout_passes=False),
        scratch_types=[
            pltpu.VMEM((KCH, CH), jnp.int32),
            pltpu.VMEM((KCH, CH), jnp.int32),
            pltpu.VMEM((KCH, CH), jnp.float32),
            pltpu.VMEM((CH, OUT), jnp.float32),
            pltpu.VMEM((CH, OUT), jnp.float32),
            pltpu.VMEM_SHARED((N, OUT), jnp.float32),
            pltpu.SemaphoreType.DMA,
            pltpu.SemaphoreType.DMA,
            pltpu.SemaphoreType.DMA,
            pltpu.SemaphoreType.DMA,
        ],
    )
    return f(xw_flat, flat_idx, dst_blk, nrm_blk)


def kernel(x, edge_index, etypes, norm, weight, w_comp, h_bias, loop_weight):
    x = x.astype(jnp.float32)
    src = edge_index[0].astype(jnp.int32)
    dst = edge_index[1].astype(jnp.int32)
    et = etypes.astype(jnp.int32)
    flat_idx = (src * R + et).reshape(E // CH, CH)
    dst_blk = dst.reshape(E // CH, CH)
    nrm_blk = norm.astype(jnp.float32).reshape(E // CH, CH)

    xw = pl.pallas_call(
        _xw_body,
        grid=(N // XBLK,),
        in_specs=[
            pl.BlockSpec(memory_space=pltpu.SMEM),
            pl.BlockSpec((B, IN, OUT), lambda i: (0, 0, 0)),
            pl.BlockSpec((XBLK, IN), lambda i: (i, 0)),
        ],
        out_specs=pl.BlockSpec((XBLK, R * OUT), lambda i: (i, 0)),
        out_shape=jax.ShapeDtypeStruct((N, R * OUT), jnp.float32),
    )(w_comp, weight, x)

    parts = _sc_scatter(xw.reshape(N * R, OUT), flat_idx, dst_blk, nrm_blk)

    h = pl.pallas_call(
        _combine_body,
        grid=(N // XBLK,),
        in_specs=[
            pl.BlockSpec((IN, OUT), lambda i: (0, 0)),
            pl.BlockSpec((1, OUT), lambda i: (0, 0)),
            pl.BlockSpec((XBLK, IN), lambda i: (i, 0)),
            pl.BlockSpec((NC, XBLK, OUT), lambda i: (0, i, 0)),
        ],
        out_specs=pl.BlockSpec((XBLK, OUT), lambda i: (i, 0)),
        out_shape=jax.ShapeDtypeStruct((N, OUT), jnp.float32),
    )(loop_weight, h_bias.reshape(1, OUT), x, parts)

    return h


# speedup vs baseline: 28.6056x; 1.0251x over previous
import functools

import jax
import jax.numpy as jnp
from jax import lax
from jax.experimental import pallas as pl
from jax.experimental.pallas import tpu as pltpu
from jax.experimental.pallas import tpu_sc as plsc

N = 10000
E = 320000
IN = 128
OUT = 128
R = 8
B = 4

NC = 2
NS = 16
LANES = 16
NW = NC * NS
EPW = E // NW
CH = 80
KCH = EPW // CH
ROWS_PT = N // NS
ZROWS = 25
XBLK = 400


def _xw_body(wc_ref, w_ref, x_ref, o_ref):
    xb = x_ref[...]
    prj = [jnp.dot(xb, w_ref[b], preferred_element_type=jnp.float32)
           for b in range(B)]
    for r in range(R):
        acc = prj[0] * wc_ref[r, 0]
        for b in range(1, B):
            acc = acc + prj[b] * wc_ref[r, b]
        o_ref[r] = acc


def _combine_body(lw_ref, bias_ref, x_ref, p_ref, o_ref):
    acc = jnp.dot(x_ref[...], lw_ref[...], preferred_element_type=jnp.float32)
    o_ref[...] = acc + p_ref[0] + p_ref[1] + bias_ref[...]


def _sc_body(xw_hbm, idx_hbm, dst_hbm, nrm_hbm, out_hbm,
             idx_v, dst_v, nrm_v, rows0, rows1, acc_sh,
             sg0, sg1, ss0, ss1):
    c = lax.axis_index("c")
    s = lax.axis_index("s")
    wid = c * NS + s

    dm0 = pltpu.async_copy(idx_hbm.at[pl.ds(wid * KCH, KCH)], idx_v, sg0)
    dm1 = pltpu.async_copy(dst_hbm.at[pl.ds(wid * KCH, KCH)], dst_v, sg1)
    dm2 = pltpu.async_copy(nrm_hbm.at[pl.ds(wid * KCH, KCH)], nrm_v, ss0)

    z16 = jnp.zeros((LANES,), jnp.float32)

    @pl.loop(0, ZROWS)
    def _(i):
        for j in range(OUT // LANES):
            rows1[i, pl.ds(j * LANES, LANES)] = z16

    for k in range(ROWS_PT // ZROWS):
        pltpu.async_copy(
            rows1.at[pl.ds(0, ZROWS)],
            acc_sh.at[pl.ds(s * ROWS_PT + k * ZROWS, ZROWS)], ss1)
    for k in range(ROWS_PT // ZROWS):
        pltpu.make_async_copy(
            rows1.at[pl.ds(0, ZROWS)],
            acc_sh.at[pl.ds(s * ROWS_PT, ZROWS)], ss1).wait()
    dm0.wait()
    dm1.wait()
    dm2.wait()
    plsc.subcore_barrier()

    def scale(buf, k):
        @plsc.parallel_loop(0, CH, unroll=4)
        def _(e):
            nb = plsc.load_gather(
                nrm_v, [jnp.full((LANES,), k, jnp.int32),
                        jnp.full((LANES,), e, jnp.int32)])
            for j in range(OUT // LANES):
                sl = (e, pl.ds(j * LANES, LANES))
                buf[sl] = buf[sl] * nb

    pltpu.async_copy(xw_hbm.at[idx_v.at[0]], rows0, sg0)

    @pl.loop(0, KCH - 1, step=2)
    def _(kk):
        @pl.when(kk > 0)
        def _():
            pltpu.make_async_copy(
                rows1, acc_sh.at[dst_v.at[kk - 1]], ss1).wait()
        dg1 = pltpu.async_copy(xw_hbm.at[idx_v.at[kk + 1]], rows1, sg1)
        pltpu.make_async_copy(xw_hbm.at[idx_v.at[kk]], rows0, sg0).wait()
        scale(rows0, kk)
        ds0 = pltpu.async_copy(rows0, acc_sh.at[dst_v.at[kk]], ss0, add=True)
        dg1.wait()
        scale(rows1, kk + 1)
        ds0.wait()
        pltpu.async_copy(xw_hbm.at[idx_v.at[kk + 2]], rows0, sg0)
        pltpu.async_copy(rows1, acc_sh.at[dst_v.at[kk + 1]], ss1, add=True)

    pltpu.make_async_copy(xw_hbm.at[idx_v.at[KCH - 1]], rows0, sg0).wait()
    scale(rows0, KCH - 1)
    pltpu.async_copy(rows0, acc_sh.at[dst_v.at[KCH - 1]], ss0, add=True)
    pltpu.make_async_copy(rows0, acc_sh.at[dst_v.at[KCH - 1]], ss0).wait()
    pltpu.make_async_copy(rows1, acc_sh.at[dst_v.at[KCH - 2]], ss1).wait()

    plsc.subcore_barrier()
    pltpu.sync_copy(acc_sh.at[pl.ds(s * ROWS_PT, ROWS_PT)],
                    out_hbm.at[c, pl.ds(s * ROWS_PT, ROWS_PT)])


def _sc_scatter(xw_flat, flat_idx, dst_blk, nrm_blk):
    mesh = plsc.VectorSubcoreMesh(core_axis_name="c", subcore_axis_name="s")
    f = pl.kernel(
        _sc_body,
        out_type=jax.ShapeDtypeStruct((NC, N, OUT), jnp.float32),
        mesh=mesh,
        compiler_params=pltpu.CompilerParams(
            use_tc_tiling_on_sc=False, needs_layout_passes=False),
        scratch_types=[
            pltpu.VMEM((KCH, CH), jnp.int32),
            pltpu.VMEM((KCH, CH), jnp.int32),
            pltpu.VMEM((KCH, CH), jnp.float32),
            pltpu.VMEM((CH, OUT), jnp.float32),
            pltpu.VMEM((CH, OUT), jnp.float32),
            pltpu.VMEM_SHARED((N, OUT), jnp.float32),
            pltpu.SemaphoreType.DMA,
            pltpu.SemaphoreType.DMA,
            pltpu.SemaphoreType.DMA,
            pltpu.SemaphoreType.DMA,
        ],
    )
    return f(xw_flat, flat_idx, dst_blk, nrm_blk)


def kernel(x, edge_index, etypes, norm, weight, w_comp, h_bias, loop_weight):
    x = x.astype(jnp.float32)
    src = edge_index[0].astype(jnp.int32)
    dst = edge_index[1].astype(jnp.int32)
    et = etypes.astype(jnp.int32)
    flat_idx = (et * N + src).reshape(E // CH, CH)
    dst_blk = dst.reshape(E // CH, CH)
    nrm_blk = norm.astype(jnp.float32).reshape(E // CH, CH)

    xw = pl.pallas_call(
        _xw_body,
        grid=(N // XBLK,),
        in_specs=[
            pl.BlockSpec(memory_space=pltpu.SMEM),
            pl.BlockSpec((B, IN, OUT), lambda i: (0, 0, 0)),
            pl.BlockSpec((XBLK, IN), lambda i: (i, 0)),
        ],
        out_specs=pl.BlockSpec((R, XBLK, OUT), lambda i: (0, i, 0)),
        out_shape=jax.ShapeDtypeStruct((R, N, OUT), jnp.float32),
    )(w_comp, weight, x)

    parts = _sc_scatter(xw.reshape(R * N, OUT), flat_idx, dst_blk, nrm_blk)

    h = pl.pallas_call(
        _combine_body,
        grid=(N // XBLK,),
        in_specs=[
            pl.BlockSpec((IN, OUT), lambda i: (0, 0)),
            pl.BlockSpec((1, OUT), lambda i: (0, 0)),
            pl.BlockSpec((XBLK, IN), lambda i: (i, 0)),
            pl.BlockSpec((NC, XBLK, OUT), lambda i: (0, i, 0)),
        ],
        out_specs=pl.BlockSpec((XBLK, OUT), lambda i: (i, 0)),
        out_shape=jax.ShapeDtypeStruct((N, OUT), jnp.float32),
    )(loop_weight, h_bias.reshape(1, OUT), x, parts)

    return h
